# baseline (speedup 1.0000x reference)
import functools

import jax
import jax.numpy as jnp
from jax import lax
from jax.experimental import pallas as pl
from jax.experimental.pallas import tpu as pltpu
from jax.experimental.pallas import tpu_sc as plsc

BETA = 0.25


def _tree_min(a):
    n = a.shape[0]
    while n > 8:
        c = n // 8
        a = functools.reduce(jnp.minimum, [a[j * c:(j + 1) * c] for j in range(8)])
        n = c
    return jnp.min(a, axis=0, keepdims=True)


def _sqnorm_body(x_ref, e_ref, xsq_ref, esq_ref):
    x = x_ref[0]
    ones = jnp.ones((1, x.shape[0]), jnp.float32)
    xsq_ref[...] = lax.dot_general(
        ones, x * x, (((1,), (0,)), ((), ())),
        preferred_element_type=jnp.float32)
    e = e_ref[...]
    esq_ref[...] = jnp.sum(e * e, axis=1, keepdims=True)


def _argmin_body(nk, bk, bm, x_ref, e_ref, xsq_ref, esq_ref, rows_ref,
                 idx_ref, minv_ref, mini_ref):
    k = pl.program_id(0)
    i = pl.program_id(1)
    x = x_ref[0]
    e = e_ref[...]

    cross_t = lax.dot_general(
        e, x, (((1,), (0,)), ((), ())),
        preferred_element_type=jnp.float32)
    xsq = xsq_ref[...]
    ch = 32
    dist_chunks = []
    macc = None
    for j in range(bk // ch):
        e_sq = esq_ref[j * ch:(j + 1) * ch, :]
        d_j = (xsq + e_sq) - 2.0 * cross_t[j * ch:(j + 1) * ch]
        dist_chunks.append(d_j)
        macc = d_j if macc is None else jnp.minimum(macc, d_j)
    m = _tree_min(macc)
    facc = None
    for j in range(bk // ch):
        s_j = jnp.where(dist_chunks[j] == m,
                        rows_ref[j * ch:(j + 1) * ch, :], float(bk))
        facc = s_j if facc is None else jnp.minimum(facc, s_j)
    first = _tree_min(facc)
    gidx = first.astype(jnp.int32) + k * bk
    sl = (slice(0, 1), pl.ds(i * bm, bm))

    @pl.when(k == 0)
    def _():
        minv_ref[sl] = m
        mini_ref[sl] = gidx

    @pl.when(k > 0)
    def _():
        better = m < minv_ref[sl]
        minv_ref[sl] = jnp.where(better, m, minv_ref[sl])
        mini_ref[sl] = jnp.where(better, gidx, mini_ref[sl])

    @pl.when(k == nk - 1)
    def _():
        idx_ref[0, 0, :] = mini_ref[sl][0]


def _nearest_code(x_lat3, embed_weight, bm=1024, bk=4096):
    b, d, hw = x_lat3.shape
    m = b * hw
    kk, _ = embed_weight.shape
    nm, nk = m // bm, kk // bk
    xsq, esq = pl.pallas_call(
        _sqnorm_body,
        grid=(b,),
        in_specs=[
            pl.BlockSpec((1, d, hw), lambda i: (i, 0, 0)),
            pl.BlockSpec((kk // b, d), lambda i: (i, 0)),
        ],
        out_specs=[
            pl.BlockSpec((1, hw), lambda i: (0, i)),
            pl.BlockSpec((kk // b, 1), lambda i: (i, 0)),
        ],
        out_shape=[
            jax.ShapeDtypeStruct((1, m), jnp.float32),
            jax.ShapeDtypeStruct((kk, 1), jnp.float32),
        ],
        compiler_params=pltpu.CompilerParams(
            dimension_semantics=("arbitrary",)),
    )(x_lat3, embed_weight)
    rows = lax.broadcasted_iota(jnp.int32, (bk, bm), 0).astype(jnp.float32)
    idx3 = pl.pallas_call(
        functools.partial(_argmin_body, nk, bk, bm),
        grid=(nk, nm),
        in_specs=[
            pl.BlockSpec((1, d, bm), lambda k, i: (i, 0, 0)),
            pl.BlockSpec((bk, d), lambda k, i: (k, 0)),
            pl.BlockSpec((1, bm), lambda k, i: (0, i)),
            pl.BlockSpec((bk, 1), lambda k, i: (k, 0)),
            pl.BlockSpec((bk, bm), lambda k, i: (0, 0)),
        ],
        out_specs=pl.BlockSpec((1, 1, bm), lambda k, i: (i, 0, 0)),
        out_shape=jax.ShapeDtypeStruct((nm, 1, bm), jnp.int32),
        scratch_shapes=[
            pltpu.VMEM((1, m), jnp.float32),
            pltpu.VMEM((1, m), jnp.int32),
        ],
        compiler_params=pltpu.CompilerParams(
            dimension_semantics=("arbitrary", "arbitrary")),
    )(x_lat3, embed_weight, xsq, esq, rows)
    return idx3.reshape(m)


def _gather_rows(embed_weight, inds):
    kk, d = embed_weight.shape
    m = inds.shape[0]
    info = plsc.get_sparse_core_info()
    nc, ns = info.num_cores, info.num_subcores
    nw = nc * ns
    b_per_w = m // nw
    nchunk = b_per_w // 128
    idx2 = inds.reshape(m // 128, 128)
    mesh = plsc.VectorSubcoreMesh(core_axis_name="c", subcore_axis_name="s")

    @functools.partial(
        pl.kernel, mesh=mesh,
        out_type=jax.ShapeDtypeStruct((m, d), jnp.float32),
        scratch_types=[
            pltpu.VMEM((nchunk, 128), jnp.int32),
            pltpu.VMEM((b_per_w, d), jnp.float32),
            pltpu.SemaphoreType.DMA,
        ],
    )
    def gather_kernel(table_hbm, idx_hbm, out_hbm, idx_v, rows_v, sem):
        wid = lax.axis_index("s") * nc + lax.axis_index("c")
        pltpu.sync_copy(idx_hbm.at[pl.ds(wid * nchunk, nchunk)], idx_v)
        copies = []
        for j in range(nchunk):
            copies.append(pltpu.async_copy(
                table_hbm.at[idx_v.at[j]],
                rows_v.at[pl.ds(j * 128, 128)], sem))
        for cp in copies:
            cp.wait()
        pltpu.sync_copy(rows_v, out_hbm.at[pl.ds(wid * b_per_w, b_per_w)])

    return gather_kernel(embed_weight, idx2)


def _assemble_body(nb, inv_n, xq_ref, x_ref, out_ref, loss_ref, acc_ref):
    b = pl.program_id(0)
    xq_t = xq_ref[0].T
    x = x_ref[0]
    out_ref[0] = x + (xq_t - x)
    t = xq_t - x
    t2 = t * t
    v = t2 * BETA + t2
    s = jnp.sum(v)

    @pl.when(b == 0)
    def _():
        acc_ref[0, 0] = s

    @pl.when(b > 0)
    def _():
        acc_ref[0, 0] = acc_ref[0, 0] + s

    @pl.when(b == nb - 1)
    def _():
        loss_ref[...] = jnp.broadcast_to(acc_ref[0, 0] * inv_n, (1, 1))


def _assemble(xq_rows, x_lat3):
    b, c, hw = x_lat3.shape
    xq3 = xq_rows.reshape(b, hw, c)
    n = b * c * hw
    out3, loss = pl.pallas_call(
        functools.partial(_assemble_body, b, 1.0 / n),
        grid=(b,),
        in_specs=[
            pl.BlockSpec((1, hw, c), lambda i: (i, 0, 0)),
            pl.BlockSpec((1, c, hw), lambda i: (i, 0, 0)),
        ],
        out_specs=[
            pl.BlockSpec((1, c, hw), lambda i: (i, 0, 0)),
            pl.BlockSpec((1, 1), lambda i: (0, 0)),
        ],
        out_shape=[
            jax.ShapeDtypeStruct((b, c, hw), jnp.float32),
            jax.ShapeDtypeStruct((1, 1), jnp.float32),
        ],
        scratch_shapes=[pltpu.SMEM((1, 1), jnp.float32)],
        compiler_params=pltpu.CompilerParams(
            dimension_semantics=("arbitrary",)),
    )(xq3, x_lat3)
    return out3, loss[0, 0]


def kernel(x_latent, embed_weight):
    b, c, h, w = x_latent.shape
    x_lat3 = x_latent.reshape(b, c, h * w)
    inds = _nearest_code(x_lat3, embed_weight)
    xq_rows = _gather_rows(embed_weight, inds)
    out3, loss = _assemble(xq_rows, x_lat3)
    return out3.reshape(b, c, h, w), loss


# speedup vs baseline: 1.0081x; 1.0081x over previous
import functools

import jax
import jax.numpy as jnp
from jax import lax
from jax.experimental import pallas as pl
from jax.experimental.pallas import tpu as pltpu
from jax.experimental.pallas import tpu_sc as plsc

BETA = 0.25


def _tree_min(a):
    n = a.shape[0]
    while n > 8:
        c = n // 8
        a = functools.reduce(jnp.minimum, [a[j * c:(j + 1) * c] for j in range(8)])
        n = c
    return jnp.min(a, axis=0, keepdims=True)


def _sqnorm_body(x_ref, e_ref, xsq_ref, esq_ref):
    x = x_ref[0]
    ones = jnp.ones((1, x.shape[0]), jnp.float32)
    xsq_ref[...] = lax.dot_general(
        ones, x * x, (((1,), (0,)), ((), ())),
        preferred_element_type=jnp.float32)
    e = e_ref[...]
    esq_ref[...] = jnp.sum(e * e, axis=1, keepdims=True)


def _argmin_body(nk, bk, bm, x_ref, e_ref, xsq_ref, esq_ref, rows_ref,
                 idx_ref, minv_ref, mini_ref):
    k = pl.program_id(0)
    i = pl.program_id(1)
    x = x_ref[0]
    e = e_ref[...]

    cross_t = lax.dot_general(
        e, x, (((1,), (0,)), ((), ())),
        preferred_element_type=jnp.float32)
    xsq = xsq_ref[...]
    ch = 32
    dist_chunks = []
    macc = None
    for j in range(bk // ch):
        e_sq = esq_ref[j * ch:(j + 1) * ch, :]
        d_j = (xsq + e_sq) - 2.0 * cross_t[j * ch:(j + 1) * ch]
        dist_chunks.append(d_j)
        macc = d_j if macc is None else jnp.minimum(macc, d_j)
    m = _tree_min(macc)
    facc = None
    for j in range(bk // ch):
        s_j = jnp.where(dist_chunks[j] == m,
                        rows_ref[j * ch:(j + 1) * ch, :], float(bk))
        facc = s_j if facc is None else jnp.minimum(facc, s_j)
    first = _tree_min(facc)
    gidx = first.astype(jnp.int32) + k * bk
    sl = (slice(0, 1), pl.ds(i * bm, bm))

    @pl.when(k == 0)
    def _():
        minv_ref[sl] = m
        mini_ref[sl] = gidx

    @pl.when(k > 0)
    def _():
        better = m < minv_ref[sl]
        minv_ref[sl] = jnp.where(better, m, minv_ref[sl])
        mini_ref[sl] = jnp.where(better, gidx, mini_ref[sl])

    @pl.when(k == nk - 1)
    def _():
        idx_ref[0, 0, :] = mini_ref[sl][0]


def _nearest_code(x_lat3, embed_weight, bm=1024, bk=2048):
    b, d, hw = x_lat3.shape
    m = b * hw
    kk, _ = embed_weight.shape
    nm, nk = m // bm, kk // bk
    xsq, esq = pl.pallas_call(
        _sqnorm_body,
        grid=(b,),
        in_specs=[
            pl.BlockSpec((1, d, hw), lambda i: (i, 0, 0)),
            pl.BlockSpec((kk // b, d), lambda i: (i, 0)),
        ],
        out_specs=[
            pl.BlockSpec((1, hw), lambda i: (0, i)),
            pl.BlockSpec((kk // b, 1), lambda i: (i, 0)),
        ],
        out_shape=[
            jax.ShapeDtypeStruct((1, m), jnp.float32),
            jax.ShapeDtypeStruct((kk, 1), jnp.float32),
        ],
        compiler_params=pltpu.CompilerParams(
            dimension_semantics=("arbitrary",)),
    )(x_lat3, embed_weight)
    rows = lax.broadcasted_iota(jnp.int32, (bk, bm), 0).astype(jnp.float32)
    idx3 = pl.pallas_call(
        functools.partial(_argmin_body, nk, bk, bm),
        grid=(nk, nm),
        in_specs=[
            pl.BlockSpec((1, d, bm), lambda k, i: (i, 0, 0)),
            pl.BlockSpec((bk, d), lambda k, i: (k, 0)),
            pl.BlockSpec((1, bm), lambda k, i: (0, i)),
            pl.BlockSpec((bk, 1), lambda k, i: (k, 0)),
            pl.BlockSpec((bk, bm), lambda k, i: (0, 0)),
        ],
        out_specs=pl.BlockSpec((1, 1, bm), lambda k, i: (i, 0, 0)),
        out_shape=jax.ShapeDtypeStruct((nm, 1, bm), jnp.int32),
        scratch_shapes=[
            pltpu.VMEM((1, m), jnp.float32),
            pltpu.VMEM((1, m), jnp.int32),
        ],
        compiler_params=pltpu.CompilerParams(
            dimension_semantics=("arbitrary", "arbitrary")),
    )(x_lat3, embed_weight, xsq, esq, rows)
    return idx3.reshape(m)


def _gather_rows(embed_weight, inds):
    kk, d = embed_weight.shape
    m = inds.shape[0]
    info = plsc.get_sparse_core_info()
    nc, ns = info.num_cores, info.num_subcores
    nw = nc * ns
    b_per_w = m // nw
    nchunk = b_per_w // 128
    idx2 = inds.reshape(m // 128, 128)
    mesh = plsc.VectorSubcoreMesh(core_axis_name="c", subcore_axis_name="s")

    @functools.partial(
        pl.kernel, mesh=mesh,
        out_type=jax.ShapeDtypeStruct((m, d), jnp.float32),
        scratch_types=[
            pltpu.VMEM((nchunk, 128), jnp.int32),
            pltpu.VMEM((b_per_w, d), jnp.float32),
            pltpu.SemaphoreType.DMA,
        ],
    )
    def gather_kernel(table_hbm, idx_hbm, out_hbm, idx_v, rows_v, sem):
        wid = lax.axis_index("s") * nc + lax.axis_index("c")
        pltpu.sync_copy(idx_hbm.at[pl.ds(wid * nchunk, nchunk)], idx_v)
        copies = []
        for j in range(nchunk):
            copies.append(pltpu.async_copy(
                table_hbm.at[idx_v.at[j]],
                rows_v.at[pl.ds(j * 128, 128)], sem))
        for cp in copies:
            cp.wait()
        pltpu.sync_copy(rows_v, out_hbm.at[pl.ds(wid * b_per_w, b_per_w)])

    return gather_kernel(embed_weight, idx2)


def _assemble_body(nb, inv_n, xq_ref, x_ref, out_ref, loss_ref, acc_ref):
    b = pl.program_id(0)
    xq_t = xq_ref[0].T
    x = x_ref[0]
    out_ref[0] = x + (xq_t - x)
    t = xq_t - x
    t2 = t * t
    v = t2 * BETA + t2
    s = jnp.sum(v)

    @pl.when(b == 0)
    def _():
        acc_ref[0, 0] = s

    @pl.when(b > 0)
    def _():
        acc_ref[0, 0] = acc_ref[0, 0] + s

    @pl.when(b == nb - 1)
    def _():
        loss_ref[...] = jnp.broadcast_to(acc_ref[0, 0] * inv_n, (1, 1))


def _assemble(xq_rows, x_lat3):
    b, c, hw = x_lat3.shape
    xq3 = xq_rows.reshape(b, hw, c)
    n = b * c * hw
    out3, loss = pl.pallas_call(
        functools.partial(_assemble_body, b, 1.0 / n),
        grid=(b,),
        in_specs=[
            pl.BlockSpec((1, hw, c), lambda i: (i, 0, 0)),
            pl.BlockSpec((1, c, hw), lambda i: (i, 0, 0)),
        ],
        out_specs=[
            pl.BlockSpec((1, c, hw), lambda i: (i, 0, 0)),
            pl.BlockSpec((1, 1), lambda i: (0, 0)),
        ],
        out_shape=[
            jax.ShapeDtypeStruct((b, c, hw), jnp.float32),
            jax.ShapeDtypeStruct((1, 1), jnp.float32),
        ],
        scratch_shapes=[pltpu.SMEM((1, 1), jnp.float32)],
        compiler_params=pltpu.CompilerParams(
            dimension_semantics=("arbitrary",)),
    )(xq3, x_lat3)
    return out3, loss[0, 0]


def kernel(x_latent, embed_weight):
    b, c, h, w = x_latent.shape
    x_lat3 = x_latent.reshape(b, c, h * w)
    inds = _nearest_code(x_lat3, embed_weight)
    xq_rows = _gather_rows(embed_weight, inds)
    out3, loss = _assemble(xq_rows, x_lat3)
    return out3.reshape(b, c, h, w), loss


# speedup vs baseline: 1.0449x; 1.0366x over previous
import functools

import jax
import jax.numpy as jnp
from jax import lax
from jax.experimental import pallas as pl
from jax.experimental.pallas import tpu as pltpu
from jax.experimental.pallas import tpu_sc as plsc

BETA = 0.25


def _tree_min(a):
    n = a.shape[0]
    while n > 8:
        c = n // 8
        a = functools.reduce(jnp.minimum, [a[j * c:(j + 1) * c] for j in range(8)])
        n = c
    return jnp.min(a, axis=0, keepdims=True)


def _argmin_body(nk, bk, bm, x_ref, e_ref, xsq_ref, esq_ref, rows_ref,
                 idx_ref, minv_ref, mini_ref):
    k = pl.program_id(0)
    i = pl.program_id(1)
    x = x_ref[...]
    e = e_ref[...]

    cross_t = lax.dot_general(
        e, x, (((1,), (1,)), ((), ())),
        preferred_element_type=jnp.float32)
    xsq = xsq_ref[...]
    ch = 32
    dist_chunks = []
    macc = None
    for j in range(bk // ch):
        e_sq = esq_ref[j * ch:(j + 1) * ch, :]
        d_j = (xsq + e_sq) - 2.0 * cross_t[j * ch:(j + 1) * ch]
        dist_chunks.append(d_j)
        macc = d_j if macc is None else jnp.minimum(macc, d_j)
    m = _tree_min(macc)
    facc = None
    for j in range(bk // ch):
        s_j = jnp.where(dist_chunks[j] == m,
                        rows_ref[j * ch:(j + 1) * ch, :], float(bk))
        facc = s_j if facc is None else jnp.minimum(facc, s_j)
    first = _tree_min(facc)
    gidx = first.astype(jnp.int32) + k * bk
    sl = (slice(0, 1), pl.ds(i * bm, bm))

    @pl.when(k == 0)
    def _():
        minv_ref[sl] = m
        mini_ref[sl] = gidx

    @pl.when(k > 0)
    def _():
        better = m < minv_ref[sl]
        minv_ref[sl] = jnp.where(better, m, minv_ref[sl])
        mini_ref[sl] = jnp.where(better, gidx, mini_ref[sl])

    @pl.when(k == nk - 1)
    def _():
        idx_ref[0, 0, :] = mini_ref[sl][0]


def _nearest_code(x_flat, embed_weight, xsq, esq, bm=1024, bk=2048):
    m, d = x_flat.shape
    kk, _ = embed_weight.shape
    nm, nk = m // bm, kk // bk
    rows = lax.broadcasted_iota(jnp.int32, (bk, bm), 0).astype(jnp.float32)
    idx3 = pl.pallas_call(
        functools.partial(_argmin_body, nk, bk, bm),
        grid=(nk, nm),
        in_specs=[
            pl.BlockSpec((bm, d), lambda k, i: (i, 0)),
            pl.BlockSpec((bk, d), lambda k, i: (k, 0)),
            pl.BlockSpec((1, bm), lambda k, i: (0, i)),
            pl.BlockSpec((bk, 1), lambda k, i: (k, 0)),
            pl.BlockSpec((bk, bm), lambda k, i: (0, 0)),
        ],
        out_specs=pl.BlockSpec((1, 1, bm), lambda k, i: (i, 0, 0)),
        out_shape=jax.ShapeDtypeStruct((nm, 1, bm), jnp.int32),
        scratch_shapes=[
            pltpu.VMEM((1, m), jnp.float32),
            pltpu.VMEM((1, m), jnp.int32),
        ],
        compiler_params=pltpu.CompilerParams(
            dimension_semantics=("arbitrary", "arbitrary")),
    )(x_flat, embed_weight, xsq, esq, rows)
    return idx3.reshape(m)


def _gather_rows(embed_weight, inds):
    kk, d = embed_weight.shape
    m = inds.shape[0]
    info = plsc.get_sparse_core_info()
    nc, ns = info.num_cores, info.num_subcores
    nw = nc * ns
    b_per_w = m // nw
    nchunk = b_per_w // 128
    idx2 = inds.reshape(m // 128, 128)
    mesh = plsc.VectorSubcoreMesh(core_axis_name="c", subcore_axis_name="s")

    @functools.partial(
        pl.kernel, mesh=mesh,
        out_type=jax.ShapeDtypeStruct((m, d), jnp.float32),
        scratch_types=[
            pltpu.VMEM((nchunk, 128), jnp.int32),
            pltpu.VMEM((b_per_w, d), jnp.float32),
            pltpu.SemaphoreType.DMA,
        ],
    )
    def gather_kernel(table_hbm, idx_hbm, out_hbm, idx_v, rows_v, sem):
        wid = lax.axis_index("s") * nc + lax.axis_index("c")
        pltpu.sync_copy(idx_hbm.at[pl.ds(wid * nchunk, nchunk)], idx_v)
        copies = []
        for j in range(nchunk):
            copies.append(pltpu.async_copy(
                table_hbm.at[idx_v.at[j]],
                rows_v.at[pl.ds(j * 128, 128)], sem))
        for cp in copies:
            cp.wait()
        pltpu.sync_copy(rows_v, out_hbm.at[pl.ds(wid * b_per_w, b_per_w)])

    return gather_kernel(embed_weight, idx2)


def _assemble_body(nb, inv_n, xq_ref, x_ref, out_ref, loss_ref, acc_ref):
    b = pl.program_id(0)
    xq_t = xq_ref[0].T
    x = x_ref[0]
    out_ref[0] = x + (xq_t - x)
    t = xq_t - x
    t2 = t * t
    v = t2 * BETA + t2
    s = jnp.sum(v)

    @pl.when(b == 0)
    def _():
        acc_ref[0, 0] = s

    @pl.when(b > 0)
    def _():
        acc_ref[0, 0] = acc_ref[0, 0] + s

    @pl.when(b == nb - 1)
    def _():
        loss_ref[...] = jnp.broadcast_to(acc_ref[0, 0] * inv_n, (1, 1))


def _assemble(xq_rows, x_lat3):
    b, c, hw = x_lat3.shape
    xq3 = xq_rows.reshape(b, hw, c)
    n = b * c * hw
    out3, loss = pl.pallas_call(
        functools.partial(_assemble_body, b, 1.0 / n),
        grid=(b,),
        in_specs=[
            pl.BlockSpec((1, hw, c), lambda i: (i, 0, 0)),
            pl.BlockSpec((1, c, hw), lambda i: (i, 0, 0)),
        ],
        out_specs=[
            pl.BlockSpec((1, c, hw), lambda i: (i, 0, 0)),
            pl.BlockSpec((1, 1), lambda i: (0, 0)),
        ],
        out_shape=[
            jax.ShapeDtypeStruct((b, c, hw), jnp.float32),
            jax.ShapeDtypeStruct((1, 1), jnp.float32),
        ],
        scratch_shapes=[pltpu.SMEM((1, 1), jnp.float32)],
        compiler_params=pltpu.CompilerParams(
            dimension_semantics=("arbitrary",)),
    )(xq3, x_lat3)
    return out3, loss[0, 0]


def kernel(x_latent, embed_weight):
    b, c, h, w = x_latent.shape
    x_lat3 = x_latent.reshape(b, c, h * w)
    x_flat3 = jnp.transpose(x_lat3, (0, 2, 1))
    x_sq = jnp.sum(x_flat3 ** 2, axis=-1, keepdims=True)
    e_sq = jnp.sum(embed_weight ** 2, axis=-1)
    x_flat = x_flat3.reshape(b * h * w, c)
    inds = _nearest_code(x_flat, embed_weight,
                         x_sq.reshape(1, b * h * w), e_sq.reshape(-1, 1))
    xq_rows = _gather_rows(embed_weight, inds)
    out3, loss = _assemble(xq_rows, x_lat3)
    return out3.reshape(b, c, h, w), loss
